# Initial kernel scaffold; baseline (speedup 1.0000x reference)
import jax
import jax.numpy as jnp
from jax.experimental import pallas as pl


def kernel(x, w):
    raise NotImplementedError("write your pallas kernel here")


# speedup vs baseline: 38.7395x; 38.7395x over previous
import functools

import jax
import jax.numpy as jnp
from jax import lax
from jax.experimental import pallas as pl
from jax.experimental.pallas import tpu as pltpu
from jax.experimental.pallas import tpu_sc as plsc

N_BASIS = 4
SEGS = 512
IN_F = 32
OUT_F = 32
BATCH = 4096

NW = 32
B_PER_W = BATCH // NW
LOOKUPS = B_PER_W * IN_F
CHUNK_B = 4
CHUNK_LK = CHUNK_B * IN_F
NCHUNK = B_PER_W // CHUNK_B
ROW_W = N_BASIS * OUT_F
LANES = 16

_mesh = plsc.VectorSubcoreMesh(core_axis_name="c", subcore_axis_name="s")


@functools.partial(
    pl.kernel,
    mesh=_mesh,
    out_type=jax.ShapeDtypeStruct((BATCH * OUT_F,), jnp.float32),
    scratch_types=[
        pltpu.VMEM((LOOKUPS,), jnp.float32),
        pltpu.VMEM((NCHUNK, CHUNK_LK), jnp.int32),
        pltpu.VMEM((LOOKUPS,), jnp.float32),
        pltpu.VMEM((LOOKUPS,), jnp.float32),
        pltpu.VMEM((LOOKUPS,), jnp.float32),
        pltpu.VMEM((LOOKUPS,), jnp.float32),
        pltpu.VMEM((CHUNK_LK, ROW_W), jnp.float32),
        pltpu.VMEM((CHUNK_LK, ROW_W), jnp.float32),
        pltpu.VMEM((LOOKUPS,), jnp.float32),
        pltpu.SemaphoreType.DMA,
        pltpu.SemaphoreType.DMA,
    ],
)
def _sc_piecewise(x_hbm, e_hbm, out_hbm, x_v, idx_v, b0_v, b1_v, b2_v, b3_v,
                  rows0, rows1, out_v, sem0, sem1):
    rows = (rows0, rows1)
    sems = (sem0, sem1)
    bas = (b0_v, b1_v, b2_v, b3_v)

    cid = lax.axis_index("c")
    sid = lax.axis_index("s")
    wid = sid * 2 + cid
    base = wid * LOOKUPS

    pltpu.sync_copy(x_hbm.at[pl.ds(base, LOOKUPS)], x_v)

    iota = lax.iota(jnp.int32, LANES)

    def seg_id(xg):
        u = xg * 256.0 + 256.0
        return jnp.minimum(jnp.maximum(u.astype(jnp.int32), 0), SEGS - 1)

    def idx_body(g, _):
        xg = x_v[pl.ds(g * LANES, LANES)]
        idv = seg_id(xg)
        ilane = (g % 2) * LANES + iota
        r = ilane * SEGS + idv
        idx_v[g // 8, pl.ds((g % 8) * LANES, LANES)] = r
        return 0

    lax.fori_loop(0, LOOKUPS // LANES, idx_body, 0)

    def fire(c, buf):
        pltpu.async_copy(e_hbm.at[idx_v.at[c]], rows[buf], sems[buf])

    fire(0, 0)
    fire(1, 1)

    def bas_body(g, _):
        xg = x_v[pl.ds(g * LANES, LANES)]
        idv = seg_id(xg)
        xm = idv.astype(jnp.float32) * (1.0 / 256.0) - 1.0
        t = (xg - xm) * 512.0 - 1.0
        d0 = t + 1.0
        d1 = t + 0.5
        d2 = t - 0.5
        d3 = t - 1.0
        b0_v[pl.ds(g * LANES, LANES)] = d1 * d2 * d3 * (-2.0 / 3.0)
        b1_v[pl.ds(g * LANES, LANES)] = d0 * d2 * d3 * (4.0 / 3.0)
        b2_v[pl.ds(g * LANES, LANES)] = d0 * d1 * d3 * (-4.0 / 3.0)
        b3_v[pl.ds(g * LANES, LANES)] = d0 * d1 * d2 * (2.0 / 3.0)
        return 0

    lax.fori_loop(0, LOOKUPS // LANES, bas_body, 0)

    def chunk_step(c, buf):
        pltpu.make_async_copy(e_hbm.at[idx_v.at[0]], rows[buf], sems[buf]).wait()
        rbuf = rows[buf]

        def b_body(bl, _):
            b = c * CHUNK_B + bl
            nbase = b * IN_F
            bv = [[bas[j][pl.ds(nbase + h * LANES, LANES)] for h in range(2)]
                  for j in range(N_BASIS)]
            accs = [jnp.zeros((LANES,), jnp.float32) for _ in range(8)]
            for i in range(IN_F):
                r = bl * IN_F + i
                for j in range(N_BASIS):
                    s = bv[j][i // LANES][i % LANES]
                    accs[2 * j] += s * rbuf[r, pl.ds(j * OUT_F, LANES)]
                    accs[2 * j + 1] += s * rbuf[r, pl.ds(j * OUT_F + LANES, LANES)]
            lo = (accs[0] + accs[2]) + (accs[4] + accs[6])
            hi = (accs[1] + accs[3]) + (accs[5] + accs[7])
            out_v[pl.ds(b * OUT_F, LANES)] = lo
            out_v[pl.ds(b * OUT_F + LANES, LANES)] = hi
            return 0

        lax.fori_loop(0, CHUNK_B, b_body, 0)

        @pl.when(c + 2 < NCHUNK)
        def _():
            fire(c + 2, buf)

    def outer(k, _):
        chunk_step(2 * k, 0)
        chunk_step(2 * k + 1, 1)
        return 0

    lax.fori_loop(0, NCHUNK // 2, outer, 0)

    pltpu.sync_copy(out_v, out_hbm.at[pl.ds(base, LOOKUPS)])


def _build_table(w):
    wt = jnp.transpose(w, (1, 2, 0))
    parts = [wt[:, j:j + 3 * SEGS:3, :] for j in range(N_BASIS)]
    return jnp.stack(parts, axis=2).reshape(IN_F * SEGS, ROW_W)


def kernel(x, w):
    e = _build_table(w)
    out_flat = _sc_piecewise(x.reshape(-1), e)
    return out_flat.reshape(BATCH, OUT_F)


# speedup vs baseline: 56.9605x; 1.4703x over previous
import functools

import jax
import jax.numpy as jnp
from jax import lax
from jax.experimental import pallas as pl
from jax.experimental.pallas import tpu as pltpu
from jax.experimental.pallas import tpu_sc as plsc

N_BASIS = 4
SEGS = 512
IN_F = 32
OUT_F = 32
BATCH = 4096

NW = 32
B_PER_W = BATCH // NW
LOOKUPS = B_PER_W * IN_F
CHUNK_B = 4
CHUNK_LK = CHUNK_B * IN_F
NCHUNK = B_PER_W // CHUNK_B
ROW_W = N_BASIS * OUT_F
LANES = 16

_mesh = plsc.VectorSubcoreMesh(core_axis_name="c", subcore_axis_name="s")


@functools.partial(
    pl.kernel,
    mesh=_mesh,
    out_type=jax.ShapeDtypeStruct((BATCH * OUT_F,), jnp.float32),
    scratch_types=[
        pltpu.VMEM((LOOKUPS,), jnp.float32),
        pltpu.VMEM((NCHUNK, CHUNK_LK), jnp.int32),
        pltpu.VMEM((LOOKUPS,), jnp.float32),
        pltpu.VMEM((LOOKUPS,), jnp.float32),
        pltpu.VMEM((LOOKUPS,), jnp.float32),
        pltpu.VMEM((LOOKUPS,), jnp.float32),
        pltpu.VMEM((CHUNK_LK, ROW_W), jnp.float32),
        pltpu.VMEM((CHUNK_LK, ROW_W), jnp.float32),
        pltpu.VMEM((LOOKUPS,), jnp.float32),
        pltpu.SemaphoreType.DMA,
        pltpu.SemaphoreType.DMA,
    ],
)
def _sc_piecewise(x_hbm, e_hbm, out_hbm, x_v, idx_v, b0_v, b1_v, b2_v, b3_v,
                  rows0, rows1, out_v, sem0, sem1):
    rows = (rows0, rows1)
    sems = (sem0, sem1)
    bas = (b0_v, b1_v, b2_v, b3_v)

    cid = lax.axis_index("c")
    sid = lax.axis_index("s")
    wid = sid * 2 + cid
    base = wid * LOOKUPS

    pltpu.sync_copy(x_hbm.at[pl.ds(base, LOOKUPS)], x_v)

    iota = lax.iota(jnp.int32, LANES)

    def seg_id(xg):
        u = xg * 256.0 + 256.0
        return jnp.minimum(jnp.maximum(u.astype(jnp.int32), 0), SEGS - 1)

    def idx_body(g, _):
        xg = x_v[pl.ds(g * LANES, LANES)]
        idv = seg_id(xg)
        ilane = (g % 2) * LANES + iota
        r = ilane * SEGS + idv
        idx_v[g // 8, pl.ds((g % 8) * LANES, LANES)] = r
        return 0

    lax.fori_loop(0, LOOKUPS // LANES, idx_body, 0)

    def fire(c, buf):
        pltpu.async_copy(e_hbm.at[idx_v.at[c]], rows[buf], sems[buf])

    fire(0, 0)
    fire(1, 1)

    def bas_body(g, _):
        xg = x_v[pl.ds(g * LANES, LANES)]
        idv = seg_id(xg)
        xm = idv.astype(jnp.float32) * (1.0 / 256.0) - 1.0
        t = (xg - xm) * 512.0 - 1.0
        d0 = t + 1.0
        d1 = t + 0.5
        d2 = t - 0.5
        d3 = t - 1.0
        b0_v[pl.ds(g * LANES, LANES)] = d1 * d2 * d3 * (-2.0 / 3.0)
        b1_v[pl.ds(g * LANES, LANES)] = d0 * d2 * d3 * (4.0 / 3.0)
        b2_v[pl.ds(g * LANES, LANES)] = d0 * d1 * d3 * (-4.0 / 3.0)
        b3_v[pl.ds(g * LANES, LANES)] = d0 * d1 * d2 * (2.0 / 3.0)
        return 0

    lax.fori_loop(0, LOOKUPS // LANES, bas_body, 0)

    def chunk_step(c, buf):
        pltpu.make_async_copy(e_hbm.at[idx_v.at[0]], rows[buf], sems[buf]).wait()
        rbuf = rows[buf]

        def b_body(bl, _):
            b = c * CHUNK_B + bl
            nbase = b * IN_F
            bv = [[bas[j][pl.ds(nbase + h * LANES, LANES)] for h in range(2)]
                  for j in range(N_BASIS)]
            accs = [jnp.zeros((LANES,), jnp.float32) for _ in range(8)]
            for i in range(IN_F):
                r = bl * IN_F + i
                for j in range(N_BASIS):
                    s = bv[j][i // LANES][i % LANES]
                    accs[2 * j] += s * rbuf[r, pl.ds(j * OUT_F, LANES)]
                    accs[2 * j + 1] += s * rbuf[r, pl.ds(j * OUT_F + LANES, LANES)]
            lo = (accs[0] + accs[2]) + (accs[4] + accs[6])
            hi = (accs[1] + accs[3]) + (accs[5] + accs[7])
            out_v[pl.ds(b * OUT_F, LANES)] = lo
            out_v[pl.ds(b * OUT_F + LANES, LANES)] = hi
            return 0

        lax.fori_loop(0, CHUNK_B, b_body, 0)

        @pl.when(c + 2 < NCHUNK)
        def _():
            fire(c + 2, buf)

    def outer(k, _):
        chunk_step(2 * k, 0)
        chunk_step(2 * k + 1, 1)
        return 0

    lax.fori_loop(0, NCHUNK // 2, outer, 0)

    pltpu.sync_copy(out_v, out_hbm.at[pl.ds(base, LOOKUPS)])


def _build_table(w):
    wp = jnp.pad(w, ((0, 0), (0, 0), (0, 2)))
    wq = wp.reshape(OUT_F, IN_F, SEGS + 1, 3)
    t = jnp.transpose(wq, (1, 2, 3, 0))
    e = jnp.concatenate([t[:, :SEGS], t[:, 1:, :1]], axis=2)
    return e.reshape(IN_F * SEGS, ROW_W)


def kernel(x, w):
    e = _build_table(w)
    out_flat = _sc_piecewise(x.reshape(-1), e)
    return out_flat.reshape(BATCH, OUT_F)


# speedup vs baseline: 66.5976x; 1.1692x over previous
import functools

import jax
import jax.numpy as jnp
from jax import lax
from jax.experimental import pallas as pl
from jax.experimental.pallas import tpu as pltpu
from jax.experimental.pallas import tpu_sc as plsc

N_BASIS = 4
SEGS = 512
IN_F = 32
OUT_F = 32
BATCH = 4096
N_COEF = (N_BASIS - 1) * SEGS + 1
TBL_ROWS = IN_F * N_COEF // 4

NW = 32
B_PER_W = BATCH // NW
LOOKUPS = B_PER_W * IN_F
ROWS_PER_B = 2 * IN_F
NBUF = 4
LANES = 16

_mesh = plsc.VectorSubcoreMesh(core_axis_name="c", subcore_axis_name="s")


@functools.partial(
    pl.kernel,
    mesh=_mesh,
    out_type=jax.ShapeDtypeStruct((BATCH * OUT_F,), jnp.float32),
    scratch_types=[
        pltpu.VMEM((LOOKUPS,), jnp.float32),
        pltpu.VMEM((B_PER_W, ROWS_PER_B), jnp.int32),
        pltpu.VMEM((LOOKUPS,), jnp.int32),
        pltpu.VMEM((LOOKUPS,), jnp.float32),
        pltpu.VMEM((LOOKUPS,), jnp.float32),
        pltpu.VMEM((LOOKUPS,), jnp.float32),
        pltpu.VMEM((LOOKUPS,), jnp.float32),
        pltpu.VMEM((ROWS_PER_B, 128), jnp.float32),
        pltpu.VMEM((ROWS_PER_B, 128), jnp.float32),
        pltpu.VMEM((ROWS_PER_B, 128), jnp.float32),
        pltpu.VMEM((ROWS_PER_B, 128), jnp.float32),
        pltpu.VMEM((LOOKUPS,), jnp.float32),
        pltpu.SemaphoreType.DMA,
        pltpu.SemaphoreType.DMA,
        pltpu.SemaphoreType.DMA,
        pltpu.SemaphoreType.DMA,
    ],
)
def _sc_piecewise(x_hbm, t_hbm, out_hbm, x_v, idx_v, off_v, b0_v, b1_v, b2_v,
                  b3_v, buf0, buf1, buf2, buf3, out_v, sem0, sem1, sem2, sem3):
    bufs = (buf0, buf1, buf2, buf3)
    sems = (sem0, sem1, sem2, sem3)
    bas = (b0_v, b1_v, b2_v, b3_v)

    cid = lax.axis_index("c")
    sid = lax.axis_index("s")
    wid = sid * 2 + cid
    base = wid * LOOKUPS

    pltpu.sync_copy(x_hbm.at[pl.ds(base, LOOKUPS)], x_v)

    iota = lax.iota(jnp.int32, LANES)

    def seg_id(xg):
        u = xg * 256.0 + 256.0
        return jnp.minimum(jnp.maximum(u.astype(jnp.int32), 0), SEGS - 1)

    def idx_body(g, _):
        xg = x_v[pl.ds(g * LANES, LANES)]
        idv = seg_id(xg)
        ilane = (g % 2) * LANES + iota
        r = ilane * N_COEF + 3 * idv
        blk = r >> 2
        b = g // 2
        col0 = (g % 2) * LANES
        idx_v[b, pl.ds(col0, LANES)] = blk
        idx_v[b, pl.ds(IN_F + col0, LANES)] = jnp.minimum(blk + 1, TBL_ROWS - 1)
        off_v[pl.ds(g * LANES, LANES)] = (r & 3) * 32
        return 0

    lax.fori_loop(0, LOOKUPS // LANES, idx_body, 0)

    def fire(b, k):
        pltpu.async_copy(t_hbm.at[idx_v.at[b]], bufs[k], sems[k])

    for p in range(NBUF):
        fire(p, p)

    def bas_body(g, _):
        xg = x_v[pl.ds(g * LANES, LANES)]
        idv = seg_id(xg)
        xm = idv.astype(jnp.float32) * (1.0 / 256.0) - 1.0
        t = (xg - xm) * 512.0 - 1.0
        d0 = t + 1.0
        d1 = t + 0.5
        d2 = t - 0.5
        d3 = t - 1.0
        b0_v[pl.ds(g * LANES, LANES)] = d1 * d2 * d3 * (-2.0 / 3.0)
        b1_v[pl.ds(g * LANES, LANES)] = d0 * d2 * d3 * (4.0 / 3.0)
        b2_v[pl.ds(g * LANES, LANES)] = d0 * d1 * d3 * (-4.0 / 3.0)
        b3_v[pl.ds(g * LANES, LANES)] = d0 * d1 * d2 * (2.0 / 3.0)
        return 0

    lax.fori_loop(0, LOOKUPS // LANES, bas_body, 0)

    def b_step(b, k):
        pltpu.make_async_copy(t_hbm.at[idx_v.at[0]], bufs[k], sems[k]).wait()
        rbuf = bufs[k]
        nbase = b * IN_F
        bv = [[bas[j][pl.ds(nbase + h * LANES, LANES)] for h in range(2)]
              for j in range(N_BASIS)]
        qv = [off_v[pl.ds(nbase + h * LANES, LANES)] for h in range(2)]
        accs = [jnp.zeros((LANES,), jnp.float32) for _ in range(8)]
        for i in range(IN_F):
            q0 = qv[i // LANES][i % LANES]
            for j in range(N_BASIS):
                s = bv[j][i // LANES][i % LANES]
                c2 = q0 + j * 32
                row = jnp.where(c2 < 128, i, IN_F + i)
                col = c2 & 127
                accs[2 * j] += s * rbuf[row, pl.ds(col, LANES)]
                accs[2 * j + 1] += s * rbuf[row, pl.ds(col + LANES, LANES)]
        lo = (accs[0] + accs[2]) + (accs[4] + accs[6])
        hi = (accs[1] + accs[3]) + (accs[5] + accs[7])
        out_v[pl.ds(b * OUT_F, LANES)] = lo
        out_v[pl.ds(b * OUT_F + LANES, LANES)] = hi

        @pl.when(b + NBUF < B_PER_W)
        def _():
            fire(b + NBUF, k)

    def outer(q, _):
        for k in range(NBUF):
            b_step(q * NBUF + k, k)
        return 0

    lax.fori_loop(0, B_PER_W // NBUF, outer, 0)

    pltpu.sync_copy(out_v, out_hbm.at[pl.ds(base, LOOKUPS)])


def kernel(x, w):
    t = jnp.transpose(w, (1, 2, 0)).reshape(TBL_ROWS, 128)
    out_flat = _sc_piecewise(x.reshape(-1), t)
    return out_flat.reshape(BATCH, OUT_F)
